# Initial kernel scaffold; baseline (speedup 1.0000x reference)
import jax
import jax.numpy as jnp
from jax.experimental import pallas as pl


def kernel(x, edge_index, edge_weight, W1, b1, W2, b2, Wf, bf):
    raise NotImplementedError("write your pallas kernel here")


# speedup vs baseline: 12.3337x; 12.3337x over previous
import functools

import jax
import jax.numpy as jnp
from jax import lax
from jax.experimental import pallas as pl
from jax.experimental.pallas import tpu as pltpu
from jax.experimental.pallas import tpu_sc as plsc

N = 10000
E = 320000
D = 128
DOUT = 16

NC = 2
NS = 16
NW = NC * NS
EW = E // NW
CH = 80
NCHUNK = EW // CH

NPAD = 10240
DEG_PER_TILE = NPAD // NS
ROWS_PER_TILE = NPAD // NS


def _mesh():
    return plsc.VectorSubcoreMesh(core_axis_name="c", subcore_axis_name="s")


@functools.partial(
    pl.kernel,
    mesh=_mesh(),
    out_type=jax.ShapeDtypeStruct((NC, NPAD), jnp.float32),
    scratch_types=[
        pltpu.VMEM_SHARED((NPAD,), jnp.float32),
        pltpu.VMEM((CH,), jnp.int32),
        pltpu.VMEM((CH,), jnp.float32),
        pltpu.VMEM((DEG_PER_TILE,), jnp.float32),
    ],
)
def _deg_sc(col_hbm, ew_hbm, out_hbm, acc, colb, ewb, zbuf):
    cid = lax.axis_index("c")
    sid = lax.axis_index("s")
    wid = cid * NS + sid

    zero16 = jnp.zeros((16,), jnp.float32)
    for i in range(DEG_PER_TILE // 16):
        zbuf[pl.ds(i * 16, 16)] = zero16
    pltpu.sync_copy(zbuf, acc.at[pl.ds(sid * DEG_PER_TILE, DEG_PER_TILE)])
    plsc.subcore_barrier()

    def body(i, carry):
        base = wid * EW + i * CH
        pltpu.sync_copy(col_hbm.at[pl.ds(base, CH)], colb)
        pltpu.sync_copy(ew_hbm.at[pl.ds(base, CH)], ewb)
        pltpu.sync_copy(ewb, acc.at[colb], add=True)
        return carry

    lax.fori_loop(0, NCHUNK, body, 0)
    plsc.subcore_barrier()
    pltpu.sync_copy(
        acc.at[pl.ds(sid * DEG_PER_TILE, DEG_PER_TILE)],
        out_hbm.at[cid, pl.ds(sid * DEG_PER_TILE, DEG_PER_TILE)],
    )


@functools.partial(
    pl.kernel,
    mesh=_mesh(),
    out_type=jax.ShapeDtypeStruct((NC, NPAD, D), jnp.float32),
    scratch_types=[
        pltpu.VMEM_SHARED((NPAD, D), jnp.float32),
        pltpu.VMEM((EW,), jnp.int32),
        pltpu.VMEM((EW,), jnp.float32),
        pltpu.VMEM((CH,), jnp.int32),
        pltpu.VMEM((CH, D), jnp.float32),
        pltpu.VMEM((ROWS_PER_TILE // 5, D), jnp.float32),
        pltpu.SemaphoreType.DMA,
    ],
)
def _spmm_sc(g_hbm, row_hbm, col_hbm, ew_hbm, out_hbm,
             acc, rowb, ewb, colb, rows, zbuf, sem):
    cid = lax.axis_index("c")
    sid = lax.axis_index("s")
    wid = cid * NS + sid
    ebase = wid * EW

    zr = ROWS_PER_TILE // 5
    zero16 = jnp.zeros((16,), jnp.float32)

    def zbody(i, carry):
        for j in range(D // 16):
            zbuf[i, pl.ds(j * 16, 16)] = zero16
        return carry

    lax.fori_loop(0, zr, zbody, 0)
    for t in range(5):
        pltpu.sync_copy(
            zbuf, acc.at[pl.ds(sid * ROWS_PER_TILE + t * zr, zr)])

    pltpu.sync_copy(row_hbm.at[pl.ds(ebase, EW)], rowb)
    pltpu.sync_copy(ew_hbm.at[pl.ds(ebase, EW)], ewb)
    plsc.subcore_barrier()

    def body(i, carry):
        base = i * CH
        pltpu.sync_copy(col_hbm.at[pl.ds(ebase + base, CH)], colb)
        pltpu.async_copy(g_hbm.at[rowb.at[pl.ds(base, CH)]], rows, sem).wait()

        def sbody(gi, c2):
            rbase = gi * 16
            ew16 = ewb[pl.ds(base + rbase, 16)]
            for k in range(16):
                w = jnp.full((16,), ew16[k], jnp.float32)
                for j in range(D // 16):
                    sl = pl.ds(j * 16, 16)
                    rows[rbase + k, sl] = rows[rbase + k, sl] * w
            return c2

        lax.fori_loop(0, CH // 16, sbody, 0)
        pltpu.sync_copy(rows, acc.at[colb], add=True)
        return carry

    lax.fori_loop(0, NCHUNK, body, 0)
    plsc.subcore_barrier()
    pltpu.sync_copy(
        acc.at[pl.ds(sid * ROWS_PER_TILE, ROWS_PER_TILE)],
        out_hbm.at[cid, pl.ds(sid * ROWS_PER_TILE, ROWS_PER_TILE)],
    )


RBLK = 1000
GRID = N // RBLK


def _fc1_body(x_ref, w1_ref, d0_ref, d1_ref, g1_ref, dinv_ref):
    deg = d0_ref[0] + d1_ref[0] + 1.0
    dinv = lax.rsqrt(deg)
    h = lax.dot_general(x_ref[...], w1_ref[...],
                        (((1,), (1,)), ((), ())),
                        preferred_element_type=jnp.float32)
    g1_ref[...] = h * dinv
    dinv_ref[...] = dinv


def _fc1(x, W1, dp):
    return pl.pallas_call(
        _fc1_body,
        grid=(GRID,),
        in_specs=[
            pl.BlockSpec((RBLK, D), lambda r: (r, 0)),
            pl.BlockSpec((D, D), lambda r: (0, 0)),
            pl.BlockSpec((1, RBLK, 1), lambda r: (0, r, 0)),
            pl.BlockSpec((1, RBLK, 1), lambda r: (1, r, 0)),
        ],
        out_specs=[
            pl.BlockSpec((RBLK, D), lambda r: (r, 0)),
            pl.BlockSpec((RBLK, 1), lambda r: (r, 0)),
        ],
        out_shape=[
            jax.ShapeDtypeStruct((N, D), jnp.float32),
            jax.ShapeDtypeStruct((N, 1), jnp.float32),
        ],
    )(x, W1, dp, dp)


def _fc2_body(s_ref, g_ref, dinv_ref, b_ref, w_ref, out_ref):
    dinv = dinv_ref[...]
    z = dinv * (s_ref[0] + s_ref[1] + g_ref[...]) + b_ref[...]
    z = jnp.maximum(z, 0.0)
    h = lax.dot_general(z, w_ref[...], (((1,), (1,)), ((), ())),
                        preferred_element_type=jnp.float32)
    out_ref[...] = h * dinv


def _fc2(s, g, dinv, b, W):
    return pl.pallas_call(
        _fc2_body,
        grid=(GRID,),
        in_specs=[
            pl.BlockSpec((NC, RBLK, D), lambda r: (0, r, 0)),
            pl.BlockSpec((RBLK, D), lambda r: (r, 0)),
            pl.BlockSpec((RBLK, 1), lambda r: (r, 0)),
            pl.BlockSpec((1, D), lambda r: (0, 0)),
            pl.BlockSpec((D, D), lambda r: (0, 0)),
        ],
        out_specs=pl.BlockSpec((RBLK, D), lambda r: (r, 0)),
        out_shape=jax.ShapeDtypeStruct((N, D), jnp.float32),
    )(s, g, dinv, b, W)


def _fc3_body(s_ref, g_ref, dinv_ref, b_ref, w_ref, bf_ref, out_ref):
    dinv = dinv_ref[...]
    z = dinv * (s_ref[0] + s_ref[1] + g_ref[...]) + b_ref[...]
    z = jnp.maximum(z, 0.0)
    h = lax.dot_general(z, w_ref[...], (((1,), (1,)), ((), ())),
                        preferred_element_type=jnp.float32)
    out_ref[...] = jax.nn.sigmoid(h + bf_ref[...])


def _fc3(s, g, dinv, b, Wf, bf):
    return pl.pallas_call(
        _fc3_body,
        grid=(GRID,),
        in_specs=[
            pl.BlockSpec((NC, RBLK, D), lambda r: (0, r, 0)),
            pl.BlockSpec((RBLK, D), lambda r: (r, 0)),
            pl.BlockSpec((RBLK, 1), lambda r: (r, 0)),
            pl.BlockSpec((1, D), lambda r: (0, 0)),
            pl.BlockSpec((DOUT, D), lambda r: (0, 0)),
            pl.BlockSpec((1, DOUT), lambda r: (0, 0)),
        ],
        out_specs=pl.BlockSpec((RBLK, DOUT), lambda r: (r, 0)),
        out_shape=jax.ShapeDtypeStruct((N, DOUT), jnp.float32),
    )(s, g, dinv, b, Wf, bf)


def kernel(x, edge_index, edge_weight, W1, b1, W2, b2, Wf, bf):
    row = edge_index[0]
    col = edge_index[1]

    degp = _deg_sc(col, edge_weight).reshape(NC, NPAD, 1)

    g1, dinv = _fc1(x, W1, degp)
    s1 = _spmm_sc(g1, row, col, edge_weight)
    g2 = _fc2(s1, g1, dinv, b1.reshape(1, D), W2)
    s2 = _spmm_sc(g2, row, col, edge_weight)
    out = _fc3(s2, g2, dinv, b2.reshape(1, D), Wf, bf.reshape(1, DOUT))
    return out
